# baseline (speedup 1.0000x reference)
import functools

import jax
import jax.numpy as jnp
from jax import lax
from jax.experimental import pallas as pl
from jax.experimental.pallas import tpu as pltpu
from jax.experimental.pallas import tpu_sc as plsc

N = 10000
E = 320000
D = 128
K = 10
ALPHA = 0.1

NC = 2
NS = 16
N2 = N // NC
CH = 32
BLK = 4
TMAX = 628
RPT = N // NS


def _linear_body(x_ref, w_ref, b_ref, o_ref):
    o_ref[...] = lax.dot_general(
        x_ref[...], w_ref[...], (((1,), (1,)), ((), ())),
        preferred_element_type=jnp.float32) + b_ref[...]


def _linear(x, W, b2):
    return pl.pallas_call(
        _linear_body,
        grid=(N // 1000,),
        in_specs=[pl.BlockSpec((1000, D), lambda i: (i, 0)),
                  pl.BlockSpec((D, D), lambda i: (0, 0)),
                  pl.BlockSpec((1, D), lambda i: (0, 0))],
        out_specs=pl.BlockSpec((1000, D), lambda i: (i, 0)),
        out_shape=jax.ShapeDtypeStruct((N, D), jnp.float32),
    )(x, W, b2)


_FINAL_SCALE = (1.0 - ALPHA) ** K


def _lsm_body(v_ref, o_ref):
    z = v_ref[...] * _FINAL_SCALE
    m = jnp.max(z, axis=1, keepdims=True)
    zs = z - m
    o_ref[...] = zs - jnp.log(jnp.sum(jnp.exp(zs), axis=1, keepdims=True))


def _logsoftmax(v):
    return pl.pallas_call(
        _lsm_body,
        grid=(N // 1000,),
        in_specs=[pl.BlockSpec((1000, D), lambda i: (i, 0))],
        out_specs=pl.BlockSpec((1000, D), lambda i: (i, 0)),
        out_shape=jax.ShapeDtypeStruct((N, D), jnp.float32),
    )(v)


_MESH = plsc.VectorSubcoreMesh(core_axis_name="c", subcore_axis_name="s")


@functools.partial(
    pl.kernel,
    out_type=jax.ShapeDtypeStruct((N, D), jnp.float32),
    mesh=_MESH,
    scratch_types=[
        pltpu.VMEM((BLK, 2, CH), jnp.int32),
        pltpu.VMEM((2, CH, D), jnp.float32),
        pltpu.VMEM_SHARED((N, D), jnp.float32),
        pltpu.VMEM_SHARED((N2 + 8, D), jnp.float32),
        pltpu.VMEM_SHARED((16,), jnp.int32),
        pltpu.SMEM((16,), jnp.int32),
        pltpu.SemaphoreType.DMA((BLK,)),
        pltpu.SemaphoreType.DMA((2,)),
        pltpu.SemaphoreType.DMA((2,)),
    ],
    compiler_params=pltpu.CompilerParams(use_tc_tiling_on_sc=False),
)
def _step(tbl_hbm, hs_hbm, idx_hbm, nblk_hbm, out_hbm,
          ir, gb, tbl, acc, nbs, nb, si, sg, ss):
    c = lax.axis_index("c")
    s = lax.axis_index("s")

    pltpu.sync_copy(tbl_hbm.at[pl.ds(s * RPT, RPT), :],
                    tbl.at[pl.ds(s * RPT, RPT), :])
    half0 = c * N2

    @pl.when(s < 8)
    def _():
        r = s * 313
        pltpu.sync_copy(hs_hbm.at[pl.ds(half0 + r, 313), :],
                        acc.at[pl.ds(r, 313), :])

    @pl.when(s >= 8)
    def _():
        r = 2504 + (s - 8) * 312
        pltpu.sync_copy(hs_hbm.at[pl.ds(half0 + r, 312), :],
                        acc.at[pl.ds(r, 312), :])

    pltpu.sync_copy(nblk_hbm, nbs)
    pltpu.sync_copy(nbs, nb)
    nblk = nb[c]

    def idx_issue(t, p):
        pltpu.async_copy(idx_hbm.at[c, s, t], ir.at[p], si.at[p])

    def idx_wait(t, p):
        pltpu.make_async_copy(idx_hbm.at[c, s, t], ir.at[p], si.at[p]).wait()

    def gather_issue(p, b):
        pltpu.async_copy(tbl.at[ir.at[p, 0]], gb.at[b], sg.at[b])

    def gather_wait(p, b):
        pltpu.make_async_copy(tbl.at[ir.at[p, 0]], gb.at[b], sg.at[b]).wait()

    def scatter_issue(p, b):
        pltpu.async_copy(gb.at[b], acc.at[ir.at[p, 1]], ss.at[b], add=True)

    def scatter_wait(p, b):
        pltpu.make_async_copy(gb.at[b], acc.at[ir.at[p, 1]], ss.at[b]).wait()

    def slot(t, r, first_block, last_block):
        b = r % 2
        gather_wait(r, b)
        scatter_issue(r, b)
        if not (first_block and r == 0):
            scatter_wait((r - 1) % BLK, (b + 1) % 2)
        if not (last_block and r == BLK - 1):
            idx_wait(t + 1, (r + 1) % BLK)
            gather_issue((r + 1) % BLK, (b + 1) % 2)
        if not (last_block and r >= BLK - 2):
            idx_issue(t + 2, (r + 2) % BLK)

    plsc.subcore_barrier()

    idx_issue(0, 0)
    idx_issue(1, 1)
    idx_wait(0, 0)
    gather_issue(0, 0)
    for r in range(BLK):
        slot(r, r, True, False)

    @pl.loop(1, nblk - 1)
    def _(j):
        t0 = j * BLK
        for r in range(BLK):
            slot(t0 + r, r, False, False)

    t0 = (nblk - 1) * BLK
    for r in range(BLK):
        slot(t0 + r, r, False, True)
    scatter_wait(BLK - 1, (BLK - 1) % 2)

    plsc.subcore_barrier()

    @pl.when(s < 8)
    def _():
        r = s * 313
        pltpu.sync_copy(acc.at[pl.ds(r, 313), :],
                        out_hbm.at[pl.ds(half0 + r, 313), :])

    @pl.when(s >= 8)
    def _():
        r = 2504 + (s - 8) * 312
        pltpu.sync_copy(acc.at[pl.ds(r, 312), :],
                        out_hbm.at[pl.ds(half0 + r, 312), :])


def kernel(x, edge_index, W, b):
    h = _linear(x, W, b.reshape(1, D))

    scales = jnp.array([ALPHA / (1.0 - ALPHA) ** (k + 1) for k in range(K)],
                       dtype=jnp.float32)
    hs = h[None] * scales[:, None, None]

    src = edge_index[0]
    dst = edge_index[1]
    m0 = dst < N2
    cnt0 = jnp.sum(m0.astype(jnp.int32))
    cnts = jnp.stack([cnt0, E - cnt0])

    cap = NS * TMAX * CH
    halves = []
    for half in range(NC):
        m = m0 if half == 0 else jnp.logical_not(m0)
        pos = jnp.cumsum(m.astype(jnp.int32)) - 1
        slots = jnp.where(m, pos, cap)
        hsrc = jnp.zeros((cap,), jnp.int32).at[slots].set(src, mode='drop')
        hdst = jnp.full((cap,), N2, jnp.int32).at[slots].set(
            dst - half * N2, mode='drop')
        hsrc = hsrc.reshape(cap // NS, NS).T.reshape(NS, TMAX, 1, CH)
        hdst = hdst.reshape(cap // NS, NS).T.reshape(NS, TMAX, 1, CH)
        halves.append(jnp.concatenate([hsrc, hdst], axis=2))
    idx = jnp.stack(halves)

    per_tile = (cnts + NS - 1) // NS
    nblk = jnp.maximum((per_tile + CH * BLK - 1) // (CH * BLK), 2)
    nblk = jnp.pad(nblk.astype(jnp.int32), (0, 14))

    v = h
    for k in range(K):
        v = _step(v, hs[k], idx, nblk)
    return _logsoftmax(v)


# speedup vs baseline: 4.6963x; 4.6963x over previous
import functools

import jax
import jax.numpy as jnp
from jax import lax
from jax.experimental import pallas as pl
from jax.experimental.pallas import tpu as pltpu
from jax.experimental.pallas import tpu_sc as plsc

N = 10000
E = 320000
D = 128
K = 10
ALPHA = 0.1

NC = 2
NS = 16
DH = D // NC
CH = 128
RING = 4
SW = 2
NCHUNK = 160
EPT = NCHUNK * CH
RPT = N // NS
RCH = 125
NRCH = RPT // RCH
LANES = 16
NBLK = NCHUNK // RING


_SCALES = tuple(ALPHA / (1.0 - ALPHA) ** (k + 1) for k in range(K))


def _linear_body(x_ref, w_ref, b_ref, o_ref, hs_ref):
    hb = lax.dot_general(
        x_ref[...], w_ref[...], (((1,), (1,)), ((), ())),
        preferred_element_type=jnp.float32) + b_ref[...]
    o_ref[...] = hb
    for k in range(K):
        hs_ref[k] = hb * _SCALES[k]


def _linear(x, W, b2):
    return pl.pallas_call(
        _linear_body,
        grid=(N // 1000,),
        in_specs=[pl.BlockSpec((1000, D), lambda i: (i, 0)),
                  pl.BlockSpec((D, D), lambda i: (0, 0)),
                  pl.BlockSpec((1, D), lambda i: (0, 0))],
        out_specs=[pl.BlockSpec((1000, D), lambda i: (i, 0)),
                   pl.BlockSpec((K, 1000, D), lambda i: (0, i, 0))],
        out_shape=[jax.ShapeDtypeStruct((N, D), jnp.float32),
                   jax.ShapeDtypeStruct((K, N, D), jnp.float32)],
    )(x, W, b2)


_FINAL_SCALE = (1.0 - ALPHA) ** K


def _lsm_body(v_ref, o_ref):
    z = v_ref[...] * _FINAL_SCALE
    m = jnp.max(z, axis=1, keepdims=True)
    zs = z - m
    o_ref[...] = zs - jnp.log(jnp.sum(jnp.exp(zs), axis=1, keepdims=True))


def _logsoftmax(v):
    return pl.pallas_call(
        _lsm_body,
        grid=(N // 1000,),
        in_specs=[pl.BlockSpec((1000, D), lambda i: (i, 0))],
        out_specs=pl.BlockSpec((1000, D), lambda i: (i, 0)),
        out_shape=jax.ShapeDtypeStruct((N, D), jnp.float32),
    )(v)


_MESH = plsc.VectorSubcoreMesh(core_axis_name="c", subcore_axis_name="s")


@functools.partial(
    pl.kernel,
    out_type=jax.ShapeDtypeStruct((N, D), jnp.float32),
    mesh=_MESH,
    scratch_types=[
        pltpu.VMEM((RING, 2, CH), jnp.int32),
        pltpu.VMEM((RING, CH, DH), jnp.float32),
        pltpu.VMEM_SHARED((N + 8, DH), jnp.float32),
        pltpu.VMEM_SHARED((N + 8, DH), jnp.float32),
        pltpu.SemaphoreType.DMA((RING,)),
        pltpu.SemaphoreType.DMA((RING,)),
        pltpu.SemaphoreType.DMA((RING,)),
    ],
    compiler_params=pltpu.CompilerParams(use_tc_tiling_on_sc=False),
)
def _propagate(h_hbm, hs_hbm, idx_hbm, out_hbm,
               ir, gb, buf_a, buf_b, si, sg, ss):
    c = lax.axis_index("c")
    s = lax.axis_index("s")
    row0 = s * RPT
    col0 = c * DH

    def stripe_init(dst_buf, k):
        if k < 0:
            src = h_hbm.at[pl.ds(row0, RPT), pl.ds(col0, DH)]
        else:
            src = hs_hbm.at[k, pl.ds(row0, RPT), pl.ds(col0, DH)]
        pltpu.sync_copy(src, dst_buf.at[pl.ds(row0, RPT), :])

    def idx_issue(t, p):
        pltpu.async_copy(idx_hbm.at[s, t], ir.at[p], si.at[p])

    def idx_wait(t, p):
        pltpu.make_async_copy(idx_hbm.at[s, t], ir.at[p], si.at[p]).wait()

    def gather_issue(table, p, b):
        pltpu.async_copy(table.at[ir.at[p, 0]], gb.at[b], sg.at[b])

    def gather_wait(table, p, b):
        pltpu.make_async_copy(table.at[ir.at[p, 0]], gb.at[b],
                              sg.at[b]).wait()

    def scatter_issue(accum, p, b):
        pltpu.async_copy(gb.at[b], accum.at[ir.at[p, 1]], ss.at[b], add=True)

    def scatter_wait(accum, p, b):
        pltpu.make_async_copy(gb.at[b], accum.at[ir.at[p, 1]],
                              ss.at[b]).wait()

    def slot(table, accum, t, r, first_block, last_block):
        b = r
        b1 = (r + 1) % RING
        bw = (r - SW) % RING
        gather_wait(table, b, b)
        scatter_issue(accum, b, b)
        if not (first_block and r < SW):
            scatter_wait(accum, bw, bw)
        if not (last_block and r == RING - 1):
            idx_wait(t + 1, b1)
            gather_issue(table, b1, b1)
        if not (last_block and r >= RING - 2):
            idx_issue(t + 2, (r + 2) % RING)

    def phase2(table, accum):
        idx_issue(0, 0)
        idx_issue(1, 1)
        idx_wait(0, 0)
        gather_issue(table, 0, 0)
        for r in range(RING):
            slot(table, accum, r, r, True, False)

        @pl.loop(1, NBLK - 1)
        def _(j):
            t0 = j * RING
            for r in range(RING):
                slot(table, accum, t0 + r, r, False, False)

        t0 = (NBLK - 1) * RING
        for r in range(RING):
            slot(table, accum, t0 + r, r, False, True)
        for w in range(SW):
            b = (RING - SW + w) % RING
            scatter_wait(accum, b, b)

    stripe_init(buf_a, -1)

    bufs = (buf_a, buf_b)
    for k in range(K):
        table = bufs[k % 2]
        accum = bufs[(k + 1) % 2]
        stripe_init(accum, k)
        plsc.subcore_barrier()
        phase2(table, accum)
        plsc.subcore_barrier()

    final = bufs[K % 2]
    pltpu.sync_copy(final.at[pl.ds(row0, RPT), :],
                    out_hbm.at[pl.ds(row0, RPT), pl.ds(col0, DH)])


def kernel(x, edge_index, W, b):
    h, hs = _linear(x, W, b.reshape(1, D))
    pad = NS * EPT - E
    src = jnp.concatenate(
        [edge_index[0], jnp.zeros((pad,), jnp.int32)]).reshape(NS, NCHUNK, 1, CH)
    dst = jnp.concatenate(
        [edge_index[1], jnp.full((pad,), N, jnp.int32)]).reshape(NS, NCHUNK, 1, CH)
    idx = jnp.concatenate([src, dst], axis=2)
    v = _propagate(h, hs, idx)
    return _logsoftmax(v)


# speedup vs baseline: 5.4780x; 1.1665x over previous
import functools

import jax
import jax.numpy as jnp
from jax import lax
from jax.experimental import pallas as pl
from jax.experimental.pallas import tpu as pltpu
from jax.experimental.pallas import tpu_sc as plsc

N = 10000
E = 320000
D = 128
K = 10
ALPHA = 0.1

NC = 2
NS = 16
DH = D // NC
CH = 128
RING = 4
SW = 2
NCHUNK = 160
EPT = NCHUNK * CH
RPT = N // NS
RCH = 125
NRCH = RPT // RCH
LANES = 16
NBLK = NCHUNK // RING


_SCALES = tuple(ALPHA / (1.0 - ALPHA) ** (k + 1) for k in range(K))


def _linear_body(x_ref, w_ref, b_ref, o_ref, hs_ref):
    hb = lax.dot_general(
        x_ref[...], w_ref[...], (((1,), (1,)), ((), ())),
        preferred_element_type=jnp.float32) + b_ref[...]
    o_ref[...] = hb
    for k in range(K):
        hs_ref[k] = hb * _SCALES[k]


def _linear(x, W, b2):
    return pl.pallas_call(
        _linear_body,
        grid=(N // 1000,),
        in_specs=[pl.BlockSpec((1000, D), lambda i: (i, 0)),
                  pl.BlockSpec((D, D), lambda i: (0, 0)),
                  pl.BlockSpec((1, D), lambda i: (0, 0))],
        out_specs=[pl.BlockSpec((1000, D), lambda i: (i, 0)),
                   pl.BlockSpec((K, 1000, D), lambda i: (0, i, 0))],
        out_shape=[jax.ShapeDtypeStruct((N, D), jnp.float32),
                   jax.ShapeDtypeStruct((K, N, D), jnp.float32)],
    )(x, W, b2)


_FINAL_SCALE = (1.0 - ALPHA) ** K


def _lsm_body(v_ref, o_ref):
    z = v_ref[...] * _FINAL_SCALE
    m = jnp.max(z, axis=1, keepdims=True)
    zs = z - m
    o_ref[...] = zs - jnp.log(jnp.sum(jnp.exp(zs), axis=1, keepdims=True))


def _logsoftmax(v):
    return pl.pallas_call(
        _lsm_body,
        grid=(N // 1000,),
        in_specs=[pl.BlockSpec((1000, D), lambda i: (i, 0))],
        out_specs=pl.BlockSpec((1000, D), lambda i: (i, 0)),
        out_shape=jax.ShapeDtypeStruct((N, D), jnp.float32),
    )(v)


_MESH = plsc.VectorSubcoreMesh(core_axis_name="c", subcore_axis_name="s")


@functools.partial(
    pl.kernel,
    out_type=jax.ShapeDtypeStruct((N, D), jnp.float32),
    mesh=_MESH,
    scratch_types=[
        pltpu.VMEM((8, 2, CH), jnp.int32),
        pltpu.VMEM((4, CH, DH), jnp.float32),
        pltpu.VMEM_SHARED((N + 8, DH), jnp.float32),
        pltpu.VMEM_SHARED((N + 8, DH), jnp.float32),
        pltpu.SemaphoreType.DMA((8,)),
        pltpu.SemaphoreType.DMA((2,)),
        pltpu.SemaphoreType.DMA((4,)),
    ],
    compiler_params=pltpu.CompilerParams(use_tc_tiling_on_sc=False),
)
def _propagate(h_hbm, hs_hbm, idx_hbm, out_hbm,
               ir, gb, buf_a, buf_b, si, sg, ss):
    c = lax.axis_index("c")
    s = lax.axis_index("s")
    row0 = s * RPT
    col0 = c * DH

    def stripe_init(dst_buf, k):
        if k < 0:
            src = h_hbm.at[pl.ds(row0, RPT), pl.ds(col0, DH)]
        else:
            src = hs_hbm.at[k, pl.ds(row0, RPT), pl.ds(col0, DH)]
        pltpu.sync_copy(src, dst_buf.at[pl.ds(row0, RPT), :])

    def idx_issue(t, p):
        pltpu.async_copy(idx_hbm.at[s, t], ir.at[p], si.at[p])

    def idx_wait(t, p):
        pltpu.make_async_copy(idx_hbm.at[s, t], ir.at[p], si.at[p]).wait()

    def gather_issue(table, p, b, g):
        pltpu.async_copy(table.at[ir.at[p, 0]], gb.at[b], sg.at[g])

    def gather_wait(table, p, b, g):
        pltpu.make_async_copy(table.at[ir.at[p, 0]], gb.at[b],
                              sg.at[g]).wait()

    def scatter_issue(accum, p, b, w):
        pltpu.async_copy(gb.at[b], accum.at[ir.at[p, 1]], ss.at[w], add=True)

    def scatter_wait(accum, p, b, w):
        pltpu.make_async_copy(gb.at[b], accum.at[ir.at[p, 1]],
                              ss.at[w]).wait()

    def slot(table, accum, t, r, first_block, last_block):
        gather_wait(table, r, r % 4, r % 2)
        scatter_issue(accum, r, r % 4, r % 4)
        if not (first_block and r < 2):
            scatter_wait(accum, (r - 2) % 8, (r - 2) % 4, (r - 2) % 4)
        if not (last_block and r >= 6):
            idx_wait(t + 2, (r + 2) % 8)
            gather_issue(table, (r + 2) % 8, (r + 2) % 4, r % 2)
        if not (last_block and r >= 4):
            idx_issue(t + 4, (r + 4) % 8)

    def phase2(table, accum):
        for p in range(4):
            idx_issue(p, p)
        idx_wait(0, 0)
        gather_issue(table, 0, 0, 0)
        idx_wait(1, 1)
        gather_issue(table, 1, 1, 1)
        for r in range(8):
            slot(table, accum, r, r, True, False)

        @pl.loop(1, NCHUNK // 8 - 1)
        def _(j):
            t0 = j * 8
            for r in range(8):
                slot(table, accum, t0 + r, r, False, False)

        t0 = NCHUNK - 8
        for r in range(8):
            slot(table, accum, t0 + r, r, False, True)
        scatter_wait(accum, 6, 2, 2)
        scatter_wait(accum, 7, 3, 3)

    stripe_init(buf_a, -1)

    bufs = (buf_a, buf_b)
    for k in range(K):
        table = bufs[k % 2]
        accum = bufs[(k + 1) % 2]
        stripe_init(accum, k)
        plsc.subcore_barrier()
        phase2(table, accum)
        plsc.subcore_barrier()

    final = bufs[K % 2]
    pltpu.sync_copy(final.at[pl.ds(row0, RPT), :],
                    out_hbm.at[pl.ds(row0, RPT), pl.ds(col0, DH)])


def kernel(x, edge_index, W, b):
    h, hs = _linear(x, W, b.reshape(1, D))
    pad = NS * EPT - E
    src = jnp.concatenate(
        [edge_index[0], jnp.zeros((pad,), jnp.int32)]).reshape(NS, NCHUNK, 1, CH)
    dst = jnp.concatenate(
        [edge_index[1], jnp.full((pad,), N, jnp.int32)]).reshape(NS, NCHUNK, 1, CH)
    idx = jnp.concatenate([src, dst], axis=2)
    v = _propagate(h, hs, idx)
    return _logsoftmax(v)
